# baseline (speedup 1.0000x reference)
import functools

import jax
import jax.numpy as jnp
from jax import lax
from jax.experimental import pallas as pl
from jax.experimental.pallas import tpu as pltpu
from jax.experimental.pallas import tpu_sc as plsc

NB = 8
NC = 19
H = 512
W = 512
NW = 32
SC_ROWS = 192
R = 8
CW = 128
CHUNKS = (SC_ROWS * W) // (R * CW * NW)
TOT = NB * CHUNKS
BROWS = NC * R
LANES = 16
ACC = NC * LANES

_mesh = plsc.VectorSubcoreMesh(core_axis_name="c", subcore_axis_name="s")


@functools.partial(
    pl.kernel,
    mesh=_mesh,
    compiler_params=pltpu.CompilerParams(needs_layout_passes=False),
    out_type=jax.ShapeDtypeStruct((NW * NB * 3, ACC), jnp.float32),
    scratch_types=[
        pltpu.VMEM((2 * BROWS, CW), jnp.float32),
        pltpu.VMEM((2 * R, CW), jnp.int32),
        pltpu.VMEM((1, ACC), jnp.float32),
        pltpu.VMEM((1, ACC), jnp.float32),
        pltpu.VMEM((1, ACC), jnp.float32),
        pltpu.SemaphoreType.DMA,
        pltpu.SemaphoreType.DMA,
    ],
)
def _dice_partials(
    inp_hbm, tgt_hbm, out_hbm, buf, tbuf, acc2, acc3, stage, sem0, sem1
):
    wid = lax.axis_index("s") * 2 + lax.axis_index("c")
    lane = lax.iota(jnp.int32, LANES)
    zeros = jnp.zeros((LANES,), jnp.float32)
    ones = jnp.ones((LANES,), jnp.float32)
    zi = jnp.zeros((LANES,), jnp.int32)

    def chunk_coords(t):
        b = t // CHUNKS
        g = wid * CHUNKS + t % CHUNKS
        h0 = (g // (W // CW)) * R
        w0 = (g % (W // CW)) * CW
        return b, h0, w0

    def fire(t, sem):
        b, h0, w0 = chunk_coords(t)
        par = t % 2
        for c in range(NC):
            pltpu.make_async_copy(
                inp_hbm.at[b, c, pl.ds(h0, R), pl.ds(w0, CW)],
                buf.at[pl.ds(par * BROWS + c * R, R), :],
                sem,
            ).start()
        pltpu.make_async_copy(
            tgt_hbm.at[b, 0, pl.ds(h0, R), pl.ds(w0, CW)],
            tbuf.at[pl.ds(par * R, R), :],
            sem,
        ).start()

    def drain(t, sem):
        par = t % 2
        pltpu.make_async_copy(
            inp_hbm.at[0, 0, pl.ds(0, BROWS), pl.ds(0, CW)],
            buf.at[pl.ds(par * BROWS, BROWS), :],
            sem,
        ).wait()
        pltpu.make_async_copy(
            tgt_hbm.at[0, 0, pl.ds(0, R), pl.ds(0, CW)],
            tbuf.at[pl.ds(par * R, R), :],
            sem,
        ).wait()

    fire(0, sem0)

    def outer(t, s1):
        par = t % 2
        reset = (t % CHUNKS) == 0
        s1 = tuple(jnp.where(reset, zeros, s) for s in s1)

        @pl.when(reset)
        def _():
            for c in range(NC):
                acc2[0, pl.ds(c * LANES, LANES)] = zeros
                acc3[0, pl.ds(c * LANES, LANES)] = zeros

        @pl.when(jnp.logical_and(t + 1 < TOT, par == 0))
        def _():
            fire(t + 1, sem1)

        @pl.when(jnp.logical_and(t + 1 < TOT, par == 1))
        def _():
            fire(t + 1, sem0)

        @pl.when(par == 0)
        def _():
            drain(t, sem0)

        @pl.when(par == 1)
        def _():
            drain(t, sem1)

        vbase = par * BROWS
        tbase = par * R

        def body(i, s1):
            r = i >> 4
            j = i & 15
            col = j * LANES
            t16 = tbuf[tbase + r, pl.ds(col, LANES)]
            w16 = lane + col
            v16 = plsc.load_gather(buf, [vbase + t16 * R + r, w16])
            idx = t16 * LANES + lane
            plsc.addupdate_scatter(acc3, [zi, idx], v16)
            plsc.addupdate_scatter(acc2, [zi, idx], ones)
            return tuple(
                s1[c] + buf[vbase + c * R + r, pl.ds(col, LANES)]
                for c in range(NC)
            )

        s1 = lax.fori_loop(0, R * CW // LANES, body, s1, unroll=2)

        @pl.when((t % CHUNKS) == CHUNKS - 1)
        def _():
            for c in range(NC):
                stage[0, pl.ds(c * LANES, LANES)] = s1[c]
            b = t // CHUNKS
            row = wid * (3 * NB) + b
            pltpu.sync_copy(stage, out_hbm.at[pl.ds(row, 1), :])
            pltpu.sync_copy(acc2, out_hbm.at[pl.ds(row + NB, 1), :])
            pltpu.sync_copy(acc3, out_hbm.at[pl.ds(row + 2 * NB, 1), :])

        return s1

    lax.fori_loop(0, TOT, outer, (zeros,) * NC)


BLK_R = 64
TC_RBLKS = (H - SC_ROWS) // BLK_R


def _tc_body(x_ref, t_ref, o_ref):
    x = x_ref[0]
    t = t_ref[0, 0]
    cls = lax.broadcasted_iota(jnp.int32, (NC, 1, 1), 0)
    m = t[None] == cls
    s1 = jnp.sum(x, axis=(1, 2))
    s2 = jnp.sum(m.astype(jnp.float32), axis=(1, 2))
    s3 = jnp.sum(jnp.where(m, x, 0.0), axis=(1, 2))
    res = jnp.stack([s1, s2, s3])[:, None, :]
    b = pl.program_id(0)
    ohb = (lax.broadcasted_iota(jnp.int32, (1, NB, 1), 1) == b).astype(
        jnp.float32
    )

    @pl.when(jnp.logical_and(b == 0, pl.program_id(1) == 0))
    def _():
        o_ref[...] = jnp.zeros_like(o_ref)

    o_ref[...] += res * ohb


_tc_partials = pl.pallas_call(
    _tc_body,
    grid=(NB, TC_RBLKS),
    in_specs=[
        pl.BlockSpec(
            (1, NC, BLK_R, W),
            lambda b, r: (b, 0, SC_ROWS // BLK_R + r, 0),
        ),
        pl.BlockSpec(
            (1, 1, BLK_R, W),
            lambda b, r: (b, 0, SC_ROWS // BLK_R + r, 0),
        ),
    ],
    out_specs=pl.BlockSpec((3, NB, NC), lambda b, r: (0, 0, 0)),
    out_shape=jax.ShapeDtypeStruct((3, NB, NC), jnp.float32),
)

SCROWS3 = NW * NB * 3


def _combine_body(sc_ref, tc_ref, o_ref):
    x = sc_ref[...]
    i304 = lax.broadcasted_iota(jnp.int32, (ACC, NC), 0)
    c19 = lax.broadcasted_iota(jnp.int32, (ACC, NC), 1)
    m_lane = (i304 // LANES == c19).astype(jnp.float32)
    y = jnp.dot(x, m_lane, preferred_element_type=jnp.float32)
    j24 = lax.broadcasted_iota(jnp.int32, (3 * NB, SCROWS3), 0)
    r768 = lax.broadcasted_iota(jnp.int32, (3 * NB, SCROWS3), 1)
    m_tile = (r768 % (3 * NB) == j24).astype(jnp.float32)
    z = jnp.dot(m_tile, y, preferred_element_type=jnp.float32)
    t = tc_ref[...]
    s1 = z[0:NB] + t[0]
    s2 = z[NB:2 * NB] + t[1]
    s3 = z[2 * NB:3 * NB] + t[2]
    r = (s3 + 1.0) / (s1 + s2 + 1.0)
    o_ref[0, 0] = jnp.float32(NC) - 0.25 * jnp.sum(r)


_combine = pl.pallas_call(
    _combine_body,
    in_specs=[
        pl.BlockSpec((SCROWS3, ACC), lambda: (0, 0)),
        pl.BlockSpec((3, NB, NC), lambda: (0, 0, 0)),
    ],
    out_specs=pl.BlockSpec(memory_space=pltpu.SMEM),
    out_shape=jax.ShapeDtypeStruct((1, 1), jnp.float32),
)


def kernel(input, target):
    tgt = target.astype(jnp.int32)
    parts_sc = _dice_partials(input, tgt)
    parts_tc = _tc_partials(input, tgt)
    return _combine(parts_sc, parts_tc)[0, 0]


# speedup vs baseline: 1.0568x; 1.0568x over previous
import functools

import jax
import jax.numpy as jnp
from jax import lax
from jax.experimental import pallas as pl
from jax.experimental.pallas import tpu as pltpu
from jax.experimental.pallas import tpu_sc as plsc

NB = 8
NC = 19
H = 512
W = 512
NW = 32
SC_ROWS = 256
R = 8
CW = 256
CHUNKS = (SC_ROWS * W) // (R * CW * NW)
TOT = NB * CHUNKS
BROWS = NC * R
LANES = 16
ACC = NC * LANES

_mesh = plsc.VectorSubcoreMesh(core_axis_name="c", subcore_axis_name="s")


@functools.partial(
    pl.kernel,
    mesh=_mesh,
    compiler_params=pltpu.CompilerParams(needs_layout_passes=False),
    out_type=jax.ShapeDtypeStruct((NW * NB * 3, ACC), jnp.float32),
    scratch_types=[
        pltpu.VMEM((2 * BROWS, CW), jnp.float32),
        pltpu.VMEM((2 * R, CW), jnp.int32),
        pltpu.VMEM((1, ACC), jnp.float32),
        pltpu.VMEM((1, ACC), jnp.float32),
        pltpu.VMEM((1, ACC), jnp.float32),
        pltpu.SemaphoreType.DMA,
        pltpu.SemaphoreType.DMA,
    ],
)
def _dice_partials(
    inp_hbm, tgt_hbm, out_hbm, buf, tbuf, acc2, acc3, stage, sem0, sem1
):
    wid = lax.axis_index("s") * 2 + lax.axis_index("c")
    lane = lax.iota(jnp.int32, LANES)
    zeros = jnp.zeros((LANES,), jnp.float32)
    ones = jnp.ones((LANES,), jnp.float32)
    zi = jnp.zeros((LANES,), jnp.int32)

    def chunk_coords(t):
        b = t // CHUNKS
        g = wid * CHUNKS + t % CHUNKS
        h0 = (g // (W // CW)) * R
        w0 = (g % (W // CW)) * CW
        return b, h0, w0

    def fire(t, sem):
        b, h0, w0 = chunk_coords(t)
        par = t % 2
        for c in range(NC):
            pltpu.make_async_copy(
                inp_hbm.at[b, c, pl.ds(h0, R), pl.ds(w0, CW)],
                buf.at[pl.ds(par * BROWS + c * R, R), :],
                sem,
            ).start()
        pltpu.make_async_copy(
            tgt_hbm.at[b, 0, pl.ds(h0, R), pl.ds(w0, CW)],
            tbuf.at[pl.ds(par * R, R), :],
            sem,
        ).start()

    def drain(t, sem):
        par = t % 2
        pltpu.make_async_copy(
            inp_hbm.at[0, 0, pl.ds(0, BROWS), pl.ds(0, CW)],
            buf.at[pl.ds(par * BROWS, BROWS), :],
            sem,
        ).wait()
        pltpu.make_async_copy(
            tgt_hbm.at[0, 0, pl.ds(0, R), pl.ds(0, CW)],
            tbuf.at[pl.ds(par * R, R), :],
            sem,
        ).wait()

    fire(0, sem0)

    def outer(t, s1):
        par = t % 2
        reset = (t % CHUNKS) == 0
        s1 = tuple(jnp.where(reset, zeros, s) for s in s1)

        @pl.when(reset)
        def _():
            for c in range(NC):
                acc2[0, pl.ds(c * LANES, LANES)] = zeros
                acc3[0, pl.ds(c * LANES, LANES)] = zeros

        @pl.when(jnp.logical_and(t + 1 < TOT, par == 0))
        def _():
            fire(t + 1, sem1)

        @pl.when(jnp.logical_and(t + 1 < TOT, par == 1))
        def _():
            fire(t + 1, sem0)

        @pl.when(par == 0)
        def _():
            drain(t, sem0)

        @pl.when(par == 1)
        def _():
            drain(t, sem1)

        vbase = par * BROWS
        tbase = par * R

        def body(i, s1):
            r = i >> 4
            j = i & 15
            col = j * LANES
            t16 = tbuf[tbase + r, pl.ds(col, LANES)]
            w16 = lane + col
            v16 = plsc.load_gather(buf, [vbase + t16 * R + r, w16])
            idx = t16 * LANES + lane
            plsc.addupdate_scatter(acc3, [zi, idx], v16)
            plsc.addupdate_scatter(acc2, [zi, idx], ones)
            return tuple(
                s1[c] + buf[vbase + c * R + r, pl.ds(col, LANES)]
                for c in range(NC)
            )

        s1 = lax.fori_loop(0, R * CW // LANES, body, s1, unroll=2)

        @pl.when((t % CHUNKS) == CHUNKS - 1)
        def _():
            for c in range(NC):
                stage[0, pl.ds(c * LANES, LANES)] = s1[c]
            b = t // CHUNKS
            row = wid * (3 * NB) + b
            pltpu.sync_copy(stage, out_hbm.at[pl.ds(row, 1), :])
            pltpu.sync_copy(acc2, out_hbm.at[pl.ds(row + NB, 1), :])
            pltpu.sync_copy(acc3, out_hbm.at[pl.ds(row + 2 * NB, 1), :])

        return s1

    lax.fori_loop(0, TOT, outer, (zeros,) * NC)


BLK_R = 64
TC_RBLKS = (H - SC_ROWS) // BLK_R


def _tc_body(x_ref, t_ref, o_ref):
    x = x_ref[0]
    t = t_ref[0, 0]
    cls = lax.broadcasted_iota(jnp.int32, (NC, 1, 1), 0)
    m = t[None] == cls
    s1 = jnp.sum(x, axis=(1, 2))
    s2 = jnp.sum(m.astype(jnp.float32), axis=(1, 2))
    s3 = jnp.sum(jnp.where(m, x, 0.0), axis=(1, 2))
    res = jnp.stack([s1, s2, s3])[:, None, :]
    b = pl.program_id(0)
    ohb = (lax.broadcasted_iota(jnp.int32, (1, NB, 1), 1) == b).astype(
        jnp.float32
    )

    @pl.when(jnp.logical_and(b == 0, pl.program_id(1) == 0))
    def _():
        o_ref[...] = jnp.zeros_like(o_ref)

    o_ref[...] += res * ohb


_tc_partials = pl.pallas_call(
    _tc_body,
    grid=(NB, TC_RBLKS),
    in_specs=[
        pl.BlockSpec(
            (1, NC, BLK_R, W),
            lambda b, r: (b, 0, SC_ROWS // BLK_R + r, 0),
        ),
        pl.BlockSpec(
            (1, 1, BLK_R, W),
            lambda b, r: (b, 0, SC_ROWS // BLK_R + r, 0),
        ),
    ],
    out_specs=pl.BlockSpec((3, NB, NC), lambda b, r: (0, 0, 0)),
    out_shape=jax.ShapeDtypeStruct((3, NB, NC), jnp.float32),
)

SCROWS3 = NW * NB * 3


def _combine_body(sc_ref, tc_ref, o_ref):
    x = sc_ref[...]
    i304 = lax.broadcasted_iota(jnp.int32, (ACC, NC), 0)
    c19 = lax.broadcasted_iota(jnp.int32, (ACC, NC), 1)
    m_lane = (i304 // LANES == c19).astype(jnp.float32)
    y = jnp.dot(x, m_lane, preferred_element_type=jnp.float32)
    j24 = lax.broadcasted_iota(jnp.int32, (3 * NB, SCROWS3), 0)
    r768 = lax.broadcasted_iota(jnp.int32, (3 * NB, SCROWS3), 1)
    m_tile = (r768 % (3 * NB) == j24).astype(jnp.float32)
    z = jnp.dot(m_tile, y, preferred_element_type=jnp.float32)
    t = tc_ref[...]
    s1 = z[0:NB] + t[0]
    s2 = z[NB:2 * NB] + t[1]
    s3 = z[2 * NB:3 * NB] + t[2]
    r = (s3 + 1.0) / (s1 + s2 + 1.0)
    o_ref[0, 0] = jnp.float32(NC) - 0.25 * jnp.sum(r)


_combine = pl.pallas_call(
    _combine_body,
    in_specs=[
        pl.BlockSpec((SCROWS3, ACC), lambda: (0, 0)),
        pl.BlockSpec((3, NB, NC), lambda: (0, 0, 0)),
    ],
    out_specs=pl.BlockSpec(memory_space=pltpu.SMEM),
    out_shape=jax.ShapeDtypeStruct((1, 1), jnp.float32),
)


def kernel(input, target):
    tgt = target.astype(jnp.int32)
    parts_sc = _dice_partials(input, tgt)
    parts_tc = _tc_partials(input, tgt)
    return _combine(parts_sc, parts_tc)[0, 0]
